# baseline (speedup 1.0000x reference)
import jax
import jax.numpy as jnp
from jax import lax
from jax.experimental import pallas as pl
from jax.experimental.pallas import tpu as pltpu
import jax.experimental.pallas.tpu_sc as plsc

C = 8192
B = 4
T = 8
N = B * T
NC = 2
NS = 16
L = 16


def _sc_body(w_hbm, x_hbm, y_hbm, out_hbm, stats_hbm,
             x_v, y_v, idx_v, row_v, stat_v, sem_in, sem_out):
    wid = lax.axis_index("s") * NC + lax.axis_index("c")

    lanes = lax.iota(jnp.int32, L)
    lane0 = lanes == 0
    zeros = jnp.zeros((L,), jnp.int32)

    pltpu.sync_copy(x_hbm, x_v)
    pltpu.sync_copy(y_hbm, y_v)
    wv = jnp.full((L,), wid, jnp.int32)
    xi = plsc.load_gather(x_v, [wv])
    yi = plsc.load_gather(y_v, [wv])

    plsc.store_scatter(idx_v, [zeros], xi, mask=lane0)
    pltpu.async_copy(w_hbm.at[idx_v], row_v, sem_in).wait()

    out_cp = pltpu.async_copy(row_v, out_hbm.at[pl.ds(wid, 1)], sem_out)

    def max_body(j, acc):
        return jnp.maximum(acc, row_v[0, pl.ds(j * L, L)])
    m_vec = lax.fori_loop(0, C // L, max_body,
                          jnp.full((L,), -jnp.inf, jnp.float32))
    m = lax.reduce_max_p.bind(m_vec, axes=(0,))

    def exp_body(j, acc):
        return acc + jnp.exp(row_v[0, pl.ds(j * L, L)] - m)
    s_vec = lax.fori_loop(0, C // L, exp_body, jnp.zeros((L,), jnp.float32))
    s = lax.reduce_sum_p.bind(s_vec, axes=(0,))

    t_vec = plsc.load_gather(row_v, [zeros, yi])

    packed = jnp.where(lanes == 0, jnp.full((L,), m, jnp.float32),
                       jnp.where(lanes == 1, jnp.full((L,), s, jnp.float32),
                                 t_vec))
    stat_v[...] = packed
    pltpu.sync_copy(stat_v, stats_hbm.at[wid])
    out_cp.wait()


_sc_call = pl.kernel(
    _sc_body,
    out_type=(jax.ShapeDtypeStruct((N, C), jnp.float32),
              jax.ShapeDtypeStruct((N, L), jnp.float32)),
    mesh=plsc.VectorSubcoreMesh(core_axis_name="c", subcore_axis_name="s"),
    compiler_params=pltpu.CompilerParams(needs_layout_passes=False),
    scratch_types=[
        pltpu.VMEM((N,), jnp.int32),
        pltpu.VMEM((N,), jnp.int32),
        pltpu.VMEM((1,), jnp.int32),
        pltpu.VMEM((1, C), jnp.float32),
        pltpu.VMEM((L,), jnp.float32),
        pltpu.SemaphoreType.DMA,
        pltpu.SemaphoreType.DMA,
    ],
)


def _tc_finish_body(stats_ref, loss_ref):
    st = stats_ref[...]
    m = st[:, 0:1]
    s = st[:, 1:2]
    t = st[:, 2:3]
    nll = jnp.log(s) + m - t
    loss_ref[...] = jnp.sum(nll, axis=0, keepdims=True) / N


def kernel(x, y, W):
    logits, stats = _sc_call(W, x.reshape(N), y.reshape(N))
    loss = pl.pallas_call(
        _tc_finish_body,
        out_shape=jax.ShapeDtypeStruct((1, 1), jnp.float32),
    )(stats)
    return logits, loss[0, 0]


# speedup vs baseline: 1.2611x; 1.2611x over previous
import jax
import jax.numpy as jnp
from jax import lax
from jax.experimental import pallas as pl
from jax.experimental.pallas import tpu as pltpu
import jax.experimental.pallas.tpu_sc as plsc

C = 8192
B = 4
T = 8
N = B * T
NC = 2
NS = 16
L = 16


def _sc_body(w_hbm, x_hbm, out_hbm, x_v, idx_v, row_v, sem):
    wid = lax.axis_index("s") * NC + lax.axis_index("c")
    lanes = lax.iota(jnp.int32, L)

    pltpu.sync_copy(x_hbm, x_v)
    xi = plsc.load_gather(x_v, [jnp.full((L,), wid, jnp.int32)])
    plsc.store_scatter(idx_v, [jnp.zeros((L,), jnp.int32)], xi,
                       mask=lanes == 0)

    pltpu.async_copy(w_hbm.at[idx_v], row_v, sem).wait()
    pltpu.sync_copy(row_v, out_hbm.at[pl.ds(wid, 1)])


_sc_gather = pl.kernel(
    _sc_body,
    out_type=jax.ShapeDtypeStruct((N, C), jnp.float32),
    mesh=plsc.VectorSubcoreMesh(core_axis_name="c", subcore_axis_name="s"),
    compiler_params=pltpu.CompilerParams(needs_layout_passes=False),
    scratch_types=[
        pltpu.VMEM((N,), jnp.int32),
        pltpu.VMEM((1,), jnp.int32),
        pltpu.VMEM((1, C), jnp.float32),
        pltpu.SemaphoreType.DMA,
    ],
)


def _tc_loss_body(xs_ref, y_ref, w_any, loss_ref, rows_v, sem):
    for i in range(N):
        pltpu.make_async_copy(
            w_any.at[pl.ds(xs_ref[i], 1)], rows_v.at[pl.ds(i, 1)], sem
        ).start()
    for i in range(N):
        pltpu.make_async_copy(
            w_any.at[pl.ds(0, 1)], rows_v.at[pl.ds(i, 1)], sem
        ).wait()

    l = rows_v[...].reshape(B, T, C)
    m = jnp.max(l, axis=2, keepdims=True)
    s = jnp.sum(jnp.exp(l - m), axis=2, keepdims=True)
    cols = lax.broadcasted_iota(jnp.int32, l.shape, 2)
    t = jnp.sum(jnp.where(cols == y_ref[...][:, :, None], l, 0.0),
                axis=2, keepdims=True)
    nll = jnp.log(s) + m - t
    loss_ref[...] = jnp.sum(nll, axis=(0, 1), keepdims=True)[:, :, 0] / N


_tc_loss = pl.pallas_call(
    _tc_loss_body,
    grid_spec=pltpu.PrefetchScalarGridSpec(
        num_scalar_prefetch=1,
        in_specs=[
            pl.BlockSpec(memory_space=pltpu.VMEM),
            pl.BlockSpec(memory_space=pl.ANY),
        ],
        out_specs=pl.BlockSpec(memory_space=pltpu.VMEM),
        scratch_shapes=[
            pltpu.VMEM((N, C), jnp.float32),
            pltpu.SemaphoreType.DMA,
        ],
    ),
    out_shape=jax.ShapeDtypeStruct((1, 1), jnp.float32),
)


def kernel(x, y, W):
    logits = _sc_gather(W, x.reshape(N))
    loss = _tc_loss(x.reshape(N), y, W)
    return logits, loss[0, 0]
